# Initial kernel scaffold; baseline (speedup 1.0000x reference)
import jax
import jax.numpy as jnp
from jax.experimental import pallas as pl


def kernel(src, index):
    raise NotImplementedError("write your pallas kernel here")


# speedup vs baseline: 4.6810x; 4.6810x over previous
import functools

import jax
import jax.numpy as jnp
from jax import lax
from jax.experimental import pallas as pl
from jax.experimental.pallas import tpu as pltpu
from jax.experimental.pallas import tpu_sc as plsc

N = 320000
D = 128
NSEG = 10000

NC = 2
NS = 16
NW = NC * NS
M = N // NW
BATCH = 125
KB = M // BATCH
KROWS = N // BATCH

_mesh = plsc.VectorSubcoreMesh(core_axis_name="c", subcore_axis_name="s")


@functools.partial(
    pl.kernel,
    mesh=_mesh,
    out_type=jax.ShapeDtypeStruct((NC, NSEG), jnp.float32),
    scratch_types=[
        pltpu.VMEM((KB, BATCH), jnp.int32),
        pltpu.VMEM((KB, BATCH), jnp.float32),
        pltpu.VMEM_SHARED((NSEG,), jnp.float32),
    ],
)
def _segsum_sc(val_hbm, idx_hbm, zeros_hbm, part_hbm, idx_v, val_v, acc):
    c = lax.axis_index("c")
    s = lax.axis_index("s")
    wid = c * NS + s
    row0 = wid * KB

    @pl.when(s == 0)
    def _():
        pltpu.sync_copy(zeros_hbm, acc)

    pltpu.sync_copy(idx_hbm.at[pl.ds(row0, KB)], idx_v)
    pltpu.sync_copy(val_hbm.at[pl.ds(row0, KB)], val_v)
    plsc.subcore_barrier()

    def step(j, carry):
        pltpu.sync_copy(val_v.at[j], acc.at[idx_v.at[j]], add=True)
        return carry

    lax.fori_loop(0, KB, step, 0)
    plsc.subcore_barrier()

    @pl.when(s == 0)
    def _():
        pltpu.sync_copy(acc, part_hbm.at[c])


def _assemble_body(p_ref, o_ref):
    p = p_ref[...]
    total = p[0] + p[1]
    colid = lax.broadcasted_iota(jnp.int32, (NSEG, D), 1)
    o_ref[...] = jnp.where(colid == 0, total[:, None], 0.0)


_assemble = pl.pallas_call(
    _assemble_body,
    out_shape=jax.ShapeDtypeStruct((NSEG, D), jnp.float32),
)


def kernel(src, index):
    col = src[:, 0].reshape(KROWS, BATCH)
    idx = index.reshape(KROWS, BATCH)
    zeros = jnp.zeros((NSEG,), jnp.float32)
    part = _segsum_sc(col, idx, zeros)
    return _assemble(part)


# speedup vs baseline: 4.7795x; 1.0210x over previous
import functools

import jax
import jax.numpy as jnp
from jax import lax
from jax.experimental import pallas as pl
from jax.experimental.pallas import tpu as pltpu
from jax.experimental.pallas import tpu_sc as plsc

N = 320000
D = 128
NSEG = 10000

NC = 2
NS = 16
NW = NC * NS
BATCH = 128
KROWS = 2560
NPAD = KROWS * BATCH - N
RPW = KROWS // NW

_mesh = plsc.VectorSubcoreMesh(core_axis_name="c", subcore_axis_name="s")


@functools.partial(
    pl.kernel,
    mesh=_mesh,
    out_type=jax.ShapeDtypeStruct((NC, NSEG), jnp.float32),
    scratch_types=[
        pltpu.VMEM((RPW, BATCH), jnp.int32),
        pltpu.VMEM((RPW, BATCH), jnp.float32),
        pltpu.VMEM_SHARED((NSEG,), jnp.float32),
    ],
)
def _segsum_sc(val_hbm, idx_hbm, zeros_hbm, part_hbm, idx_v, val_v, acc):
    c = lax.axis_index("c")
    s = lax.axis_index("s")
    wid = c * NS + s
    row0 = wid * RPW

    @pl.when(s == 0)
    def _():
        pltpu.sync_copy(zeros_hbm, acc)

    pltpu.sync_copy(idx_hbm.at[pl.ds(row0, RPW)], idx_v)
    pltpu.sync_copy(val_hbm.at[pl.ds(row0, RPW)], val_v)

    plsc.subcore_barrier()

    def step(j, carry):
        pltpu.sync_copy(val_v.at[j], acc.at[idx_v.at[j]], add=True)
        return carry

    lax.fori_loop(0, RPW, step, 0)
    plsc.subcore_barrier()

    @pl.when(s == 0)
    def _():
        pltpu.sync_copy(acc, part_hbm.at[c])


def _assemble_body(p_ref, o_ref):
    p = p_ref[...]
    total = p[0] + p[1]
    colid = lax.broadcasted_iota(jnp.int32, (NSEG, D), 1)
    o_ref[...] = jnp.where(colid == 0, total[:, None], 0.0)


_assemble = pl.pallas_call(
    _assemble_body,
    out_shape=jax.ShapeDtypeStruct((NSEG, D), jnp.float32),
)


def kernel(src, index):
    col = jnp.pad(src[:, 0], (0, NPAD)).reshape(KROWS, BATCH)
    idx = jnp.pad(index, (0, NPAD)).reshape(KROWS, BATCH)
    zeros = jnp.zeros((NSEG,), jnp.float32)
    part = _segsum_sc(col, idx, zeros)
    return _assemble(part)


# speedup vs baseline: 5.2435x; 1.0971x over previous
import functools

import jax
import jax.numpy as jnp
from jax import lax
from jax.experimental import pallas as pl
from jax.experimental.pallas import tpu as pltpu
from jax.experimental.pallas import tpu_sc as plsc

N = 320000
D = 128
NSEG = 10000

NC = 2
NS = 16
NW = NC * NS
BATCH = 128
KROWS = 2560
NP = KROWS * BATCH
RPW = KROWS // NW
NSEGP = NSEG + BATCH

_mesh = plsc.VectorSubcoreMesh(core_axis_name="c", subcore_axis_name="s")


@functools.partial(
    pl.kernel,
    mesh=_mesh,
    out_type=jax.ShapeDtypeStruct((NC, NSEGP), jnp.float32),
    scratch_types=[
        pltpu.VMEM((RPW, BATCH), jnp.int32),
        pltpu.VMEM((RPW, BATCH), jnp.int32),
        pltpu.VMEM((RPW, BATCH), jnp.float32),
        pltpu.VMEM_SHARED((NSEGP,), jnp.float32),
        pltpu.SemaphoreType.DMA,
    ],
)
def _segsum_sc(src_hbm, idx_hbm, gidx_hbm, zeros_hbm, part_hbm,
               idx_v, gidx_v, vbuf, acc, gsem):
    c = lax.axis_index("c")
    s = lax.axis_index("s")
    wid = c * NS + s
    row0 = wid * RPW

    pltpu.sync_copy(idx_hbm.at[pl.ds(row0, RPW)], idx_v)
    pltpu.sync_copy(gidx_hbm.at[pl.ds(row0, RPW)], gidx_v)

    def fire(j, carry):
        pltpu.async_copy(src_hbm.at[gidx_v.at[j]], vbuf.at[j], gsem)
        return carry

    lax.fori_loop(0, RPW, fire, 0)

    @pl.when(s == 0)
    def _():
        pltpu.sync_copy(zeros_hbm, acc)

    def drain(j, carry):
        pltpu.make_async_copy(src_hbm.at[gidx_v.at[j]], vbuf.at[j], gsem).wait()
        return carry

    lax.fori_loop(0, RPW, drain, 0)
    plsc.subcore_barrier()

    def scat(j, carry):
        pltpu.sync_copy(vbuf.at[j], acc.at[idx_v.at[j]], add=True)
        return carry

    lax.fori_loop(0, RPW, scat, 0)
    plsc.subcore_barrier()

    @pl.when(s == 0)
    def _():
        pltpu.sync_copy(acc, part_hbm.at[c])


def _assemble_body(p_ref, o_ref):
    p = p_ref[...]
    total = (p[0] + p[1])[:NSEG]
    colid = lax.broadcasted_iota(jnp.int32, (NSEG, D), 1)
    o_ref[...] = jnp.where(colid == 0, total[:, None], 0.0)


_assemble = pl.pallas_call(
    _assemble_body,
    out_shape=jax.ShapeDtypeStruct((NSEG, D), jnp.float32),
)


def kernel(src, index):
    srcf = src.reshape(N * D)
    pad_idx = (NSEG + jnp.arange(NP - N, dtype=jnp.int32) % BATCH)
    idx = jnp.concatenate([index, pad_idx]).reshape(KROWS, BATCH)
    gidx = (jnp.minimum(jnp.arange(NP), N - 1) * D).astype(jnp.int32)
    gidx = gidx.reshape(KROWS, BATCH)
    zeros = jnp.zeros((NSEGP,), jnp.float32)
    part = _segsum_sc(srcf, idx, gidx, zeros)
    return _assemble(part)


# speedup vs baseline: 8.8145x; 1.6810x over previous
import functools

import jax
import jax.numpy as jnp
from jax import lax
from jax.experimental import pallas as pl
from jax.experimental.pallas import tpu as pltpu
from jax.experimental.pallas import tpu_sc as plsc

N = 320000
D = 128
NSEG = 10000

NC = 2
NS = 16
NW = NC * NS
BATCH = 128
NB = N // BATCH
KROWS = 2560
NP = KROWS * BATCH
RPW = KROWS // NW
NBLAST = NB - (NW - 1) * RPW

_mesh = plsc.VectorSubcoreMesh(core_axis_name="c", subcore_axis_name="s")


@functools.partial(
    pl.kernel,
    mesh=_mesh,
    out_type=jax.ShapeDtypeStruct((NC, NSEG), jnp.float32),
    scratch_types=[
        pltpu.VMEM((RPW, BATCH), jnp.int32),
        pltpu.VMEM((RPW, BATCH), jnp.int32),
        pltpu.VMEM((RPW, BATCH), jnp.float32),
        pltpu.VMEM_SHARED((NSEG,), jnp.float32),
        pltpu.SemaphoreType.DMA,
    ],
)
def _segsum_sc(src_hbm, idx_hbm, gidx_hbm, zeros_hbm, part_hbm,
               idx_v, gidx_v, vbuf, acc, gsem):
    c = lax.axis_index("c")
    s = lax.axis_index("s")
    wid = c * NS + s
    row0 = wid * RPW
    nb = jnp.where(wid == NW - 1, NBLAST, RPW)

    pltpu.sync_copy(idx_hbm.at[pl.ds(row0, RPW)], idx_v)
    pltpu.sync_copy(gidx_hbm.at[pl.ds(row0, RPW)], gidx_v)

    def fire(j, carry):
        pltpu.async_copy(src_hbm.at[gidx_v.at[j]], vbuf.at[j], gsem)
        return carry

    lax.fori_loop(0, nb, fire, 0)

    @pl.when(s == 0)
    def _():
        pltpu.sync_copy(zeros_hbm, acc)

    def drain(j, carry):
        pltpu.make_async_copy(src_hbm.at[gidx_v.at[j]], vbuf.at[j], gsem).wait()
        return carry

    lax.fori_loop(0, nb, drain, 0)
    plsc.subcore_barrier()

    def scat(j, carry):
        pltpu.sync_copy(vbuf.at[j], acc.at[idx_v.at[j]], add=True)
        return carry

    lax.fori_loop(0, nb, scat, 0)
    plsc.subcore_barrier()

    @pl.when(s == 0)
    def _():
        pltpu.sync_copy(acc, part_hbm.at[c])


def _assemble_body(p_ref, o_ref):
    p = p_ref[...]
    total = p[0] + p[1]
    colid = lax.broadcasted_iota(jnp.int32, (NSEG, D), 1)
    o_ref[...] = jnp.where(colid == 0, total[:, None], 0.0)


_assemble = pl.pallas_call(
    _assemble_body,
    out_shape=jax.ShapeDtypeStruct((NSEG, D), jnp.float32),
)


def kernel(src, index):
    srcf = src.reshape(N * D)
    idx = jnp.pad(index, (0, NP - N)).reshape(KROWS, BATCH)
    gidx = (jnp.arange(NP, dtype=jnp.int32) * D).reshape(KROWS, BATCH)
    zeros = jnp.zeros((NSEG,), jnp.float32)
    part = _segsum_sc(srcf, idx, gidx, zeros)
    return _assemble(part)


# speedup vs baseline: 9.9185x; 1.1252x over previous
import functools

import jax
import jax.numpy as jnp
from jax import lax
from jax.experimental import pallas as pl
from jax.experimental.pallas import tpu as pltpu
from jax.experimental.pallas import tpu_sc as plsc

N = 320000
D = 128
NSEG = 10000

NC = 2
NS = 16
NW = NC * NS
BATCH = 128
NB = N // BATCH
KROWS = 2560
NP = KROWS * BATCH
RPW = KROWS // NW
NBLAST = NB - (NW - 1) * RPW
CH = 4
NCHUNK = RPW // CH
ZCH = 640

_mesh = plsc.VectorSubcoreMesh(core_axis_name="c", subcore_axis_name="s")


@functools.partial(
    pl.kernel,
    mesh=_mesh,
    out_type=jax.ShapeDtypeStruct((NC, NSEG), jnp.float32),
    scratch_types=[
        pltpu.VMEM((RPW, BATCH), jnp.int32),
        pltpu.VMEM((RPW, BATCH), jnp.int32),
        pltpu.VMEM((RPW, BATCH), jnp.float32),
        pltpu.VMEM((ZCH,), jnp.float32),
        pltpu.VMEM_SHARED((NSEG,), jnp.float32),
        pltpu.SemaphoreType.DMA,
        pltpu.SemaphoreType.DMA,
        pltpu.SemaphoreType.DMA,
    ],
)
def _segsum_sc(src_hbm, idx_hbm, part_hbm,
               idx_v, gidx_v, vbuf, zbuf, acc, semA, semB, isem):
    c = lax.axis_index("c")
    s = lax.axis_index("s")
    wid = c * NS + s
    row0 = wid * RPW
    nb = jnp.where(wid == NW - 1, NBLAST, RPW)

    pltpu.async_copy(idx_hbm.at[pl.ds(row0, RPW)], idx_v, isem)

    iota = lax.iota(jnp.int32, 16)
    b0 = row0 * (BATCH * D)
    lane = iota * D

    def gen(j, carry):
        base = b0 + j * (BATCH * D)
        for m in range(8):
            gidx_v[j, pl.ds(16 * m, 16)] = base + 16 * D * m + lane
        return carry

    lax.fori_loop(0, RPW, gen, 0)

    def fire_chunk(t, sem):
        for k in range(CH):
            j = t * CH + k

            @pl.when(j < nb)
            def _():
                pltpu.async_copy(src_hbm.at[gidx_v.at[j]], vbuf.at[j], sem)

    def drain_chunk(t, sem):
        for k in range(CH):
            j = t * CH + k

            @pl.when(j < nb)
            def _():
                pltpu.make_async_copy(
                    src_hbm.at[gidx_v.at[j]], vbuf.at[j], sem).wait()

    def scat_chunk(t):
        for k in range(CH):
            j = t * CH + k

            @pl.when(j < nb)
            def _():
                pltpu.sync_copy(vbuf.at[j], acc.at[idx_v.at[j]], add=True)

    fire_chunk(0, semA)

    zv = jnp.zeros((16,), jnp.float32)

    def zstep(i, carry):
        zbuf[pl.ds(16 * i, 16)] = zv
        return carry

    lax.fori_loop(0, ZCH // 16, zstep, 0)

    @pl.when(s < NS - 1)
    def _():
        pltpu.sync_copy(zbuf, acc.at[pl.ds(s * ZCH, ZCH)])

    @pl.when(s == NS - 1)
    def _():
        pltpu.sync_copy(zbuf.at[pl.ds(0, NSEG - (NS - 1) * ZCH)],
                        acc.at[pl.ds((NS - 1) * ZCH, NSEG - (NS - 1) * ZCH)])

    pltpu.make_async_copy(idx_hbm.at[pl.ds(row0, RPW)], idx_v, isem).wait()
    plsc.subcore_barrier()

    def body(u, carry):
        t0 = 2 * u
        fire_chunk(t0 + 1, semB)
        drain_chunk(t0, semA)
        scat_chunk(t0)
        fire_chunk(t0 + 2, semA)
        drain_chunk(t0 + 1, semB)
        scat_chunk(t0 + 1)
        return carry

    lax.fori_loop(0, NCHUNK // 2, body, 0)
    plsc.subcore_barrier()

    @pl.when(s == 0)
    def _():
        pltpu.sync_copy(acc, part_hbm.at[c])


def _assemble_body(p_ref, o_ref):
    p = p_ref[...]
    total = p[0] + p[1]
    colid = lax.broadcasted_iota(jnp.int32, (NSEG, D), 1)
    o_ref[...] = jnp.where(colid == 0, total[:, None], 0.0)


_assemble = pl.pallas_call(
    _assemble_body,
    out_shape=jax.ShapeDtypeStruct((NSEG, D), jnp.float32),
)


def kernel(src, index):
    srcf = src.reshape(N * D)
    idx = jnp.pad(index, (0, NP - N)).reshape(KROWS, BATCH)
    part = _segsum_sc(srcf, idx)
    return _assemble(part)


# speedup vs baseline: 10.5768x; 1.0664x over previous
import functools

import jax
import jax.numpy as jnp
from jax import lax
from jax.experimental import pallas as pl
from jax.experimental.pallas import tpu as pltpu
from jax.experimental.pallas import tpu_sc as plsc

N = 320000
D = 128
NSEG = 10000

NC = 2
NS = 16
NW = NC * NS
BATCH = 128
NB = N // BATCH
KROWS = 2560
NP = KROWS * BATCH
RPW = KROWS // NW
NBLAST = NB - (NW - 1) * RPW
CH = 4
NCHUNK = RPW // CH
ZCH = 640

_mesh = plsc.VectorSubcoreMesh(core_axis_name="c", subcore_axis_name="s")


@functools.partial(
    pl.kernel,
    mesh=_mesh,
    out_type=jax.ShapeDtypeStruct((NC, NSEG), jnp.float32),
    scratch_types=[
        pltpu.VMEM((RPW, BATCH), jnp.int32),
        pltpu.VMEM((RPW, BATCH), jnp.int32),
        pltpu.VMEM((RPW, BATCH), jnp.float32),
        pltpu.VMEM((ZCH,), jnp.float32),
        pltpu.VMEM_SHARED((NSEG,), jnp.float32),
        pltpu.SemaphoreType.DMA,
        pltpu.SemaphoreType.DMA,
        pltpu.SemaphoreType.DMA,
        pltpu.SemaphoreType.DMA,
    ],
)
def _segsum_sc(src_hbm, idx_hbm, part_hbm,
               idx_v, gidx_v, vbuf, zbuf, acc, semA, semB, isem, ssem):
    c = lax.axis_index("c")
    s = lax.axis_index("s")
    wid = c * NS + s
    row0 = wid * RPW
    last = wid == NW - 1
    nb = jnp.where(last, NBLAST, RPW)

    pltpu.async_copy(idx_hbm.at[pl.ds(row0, RPW)], idx_v, isem)

    iota = lax.iota(jnp.int32, 16)
    b0 = row0 * (BATCH * D)
    lane = iota * D

    def gen(j, carry):
        base = b0 + j * (BATCH * D)
        for m in range(8):
            gidx_v[j, pl.ds(16 * m, 16)] = base + 16 * D * m + lane
        return carry

    lax.fori_loop(0, RPW, gen, 0)

    def fire_chunk(t, sem):
        for k in range(CH):
            j = t * CH + k

            @pl.when(j < nb)
            def _():
                pltpu.async_copy(src_hbm.at[gidx_v.at[j]], vbuf.at[j], sem)

    def drain_chunk(t, sem):
        for k in range(CH):
            j = t * CH + k

            @pl.when(j < nb)
            def _():
                pltpu.make_async_copy(
                    src_hbm.at[gidx_v.at[j]], vbuf.at[j], sem).wait()

    def scat_chunk(t):
        for k in range(CH):
            j = t * CH + k

            @pl.when(j < nb)
            def _():
                pltpu.async_copy(vbuf.at[j], acc.at[idx_v.at[j]], ssem,
                                 add=True)

    fire_chunk(0, semA)

    zv = jnp.zeros((16,), jnp.float32)

    def zstep(i, carry):
        zbuf[pl.ds(16 * i, 16)] = zv
        return carry

    lax.fori_loop(0, ZCH // 16, zstep, 0)

    @pl.when(s < NS - 1)
    def _():
        pltpu.sync_copy(zbuf, acc.at[pl.ds(s * ZCH, ZCH)])

    @pl.when(s == NS - 1)
    def _():
        pltpu.sync_copy(zbuf.at[pl.ds(0, NSEG - (NS - 1) * ZCH)],
                        acc.at[pl.ds((NS - 1) * ZCH, NSEG - (NS - 1) * ZCH)])

    pltpu.make_async_copy(idx_hbm.at[pl.ds(row0, RPW)], idx_v, isem).wait()
    plsc.subcore_barrier()

    def body(u, carry):
        t0 = 2 * u
        fire_chunk(t0 + 1, semB)
        drain_chunk(t0, semA)
        scat_chunk(t0)
        fire_chunk(t0 + 2, semA)
        drain_chunk(t0 + 1, semB)
        scat_chunk(t0 + 1)
        return carry

    lax.fori_loop(0, NCHUNK // 2, body, 0)

    def sdrain(j, carry):
        @pl.when(j < nb)
        def _():
            pltpu.make_async_copy(vbuf.at[j], acc.at[idx_v.at[j]],
                                  ssem).wait()
        return carry

    lax.fori_loop(0, RPW, sdrain, 0)
    plsc.subcore_barrier()

    @pl.when(s == 0)
    def _():
        pltpu.sync_copy(acc, part_hbm.at[c])


def _assemble_body(p_ref, o_ref):
    p = p_ref[...]
    total = p[0] + p[1]
    colid = lax.broadcasted_iota(jnp.int32, (NSEG, D), 1)
    o_ref[...] = jnp.where(colid == 0, total[:, None], 0.0)


_assemble = pl.pallas_call(
    _assemble_body,
    out_shape=jax.ShapeDtypeStruct((NSEG, D), jnp.float32),
)


def kernel(src, index):
    srcf = src.reshape(N * D)
    idx = jnp.pad(index, (0, NP - N)).reshape(KROWS, BATCH)
    part = _segsum_sc(srcf, idx)
    return _assemble(part)
